# Initial kernel scaffold; baseline (speedup 1.0000x reference)
import functools

import jax
import jax.numpy as jnp
from jax import lax
from jax.experimental import pallas as pl
from jax.experimental.pallas import tpu as pltpu
from jax.experimental.pallas import tpu_sc as plsc

VOCAB = 100000
EMBED = 64
BATCH = 4096
HIST = 50
N = BATCH * HIST
NC, NS, L = 2, 16, 16
NW = NC * NS
CHUNK = 128
NPW = N // NW
NCHUNK = NPW // CHUNK
VPR = EMBED // L

C0 = 0.99949463
C1 = -0.491904
C2 = 0.28946795
C3 = -0.13606202
C4 = 0.03216066


def _softplus_chunk(buf):
    @pl.loop(0, CHUNK)
    def _row(i):
        for k in range(VPR):
            sl = (i, pl.ds(k * L, L))
            x = buf[sl]
            r = jnp.maximum(x, 0.0)
            t = jnp.exp(-jnp.abs(x))
            p = C4
            p = p * t + C3
            p = p * t + C2
            p = p * t + C1
            p = p * t + C0
            buf[sl] = r + t * p


def _sc_body(emb_hbm, word_hbm, out_hbm, idx_v, buf0, buf1,
             gsem0, gsem1, ssem0, ssem1):
    c = lax.axis_index("c")
    s = lax.axis_index("s")
    wid = c * NS + s
    row0 = wid * NCHUNK

    pltpu.sync_copy(word_hbm.at[pl.ds(row0, NCHUNK)], idx_v)

    def start_gather(chunk, buf, sem):
        return pltpu.async_copy(emb_hbm.at[idx_v.at[chunk]], buf, sem)

    def wait_gather(chunk, buf, sem):
        pltpu.make_async_copy(emb_hbm.at[idx_v.at[chunk]], buf, sem).wait()

    def out_slice(chunk):
        return out_hbm.at[pl.ds((row0 + chunk) * CHUNK, CHUNK)]

    def start_store(chunk, buf, sem):
        return pltpu.async_copy(buf, out_slice(chunk), sem)

    def wait_store(chunk, buf, sem):
        pltpu.make_async_copy(buf, out_slice(chunk), sem).wait()

    start_gather(0, buf0, gsem0)

    @pl.loop(0, NCHUNK, step=2)
    def _pair(j):
        c0 = j
        c1 = j + 1

        @pl.when(c0 > 0)
        def _():
            wait_store(c1 - 2, buf1, ssem1)

        start_gather(c1, buf1, gsem1)
        wait_gather(c0, buf0, gsem0)
        _softplus_chunk(buf0)
        start_store(c0, buf0, ssem0)

        @pl.when(c0 + 2 < NCHUNK)
        def _():
            wait_store(c0, buf0, ssem0)
            start_gather(c0 + 2, buf0, gsem0)

        wait_gather(c1, buf1, gsem1)
        _softplus_chunk(buf1)
        start_store(c1, buf1, ssem1)

    wait_store(NCHUNK - 2, buf0, ssem0)
    wait_store(NCHUNK - 1, buf1, ssem1)


_sc_call = functools.partial(
    pl.kernel,
    out_type=jax.ShapeDtypeStruct((N, EMBED), jnp.float32),
    mesh=plsc.VectorSubcoreMesh(
        core_axis_name="c", subcore_axis_name="s",
        num_cores=NC, num_subcores=NS),
    scratch_types=[
        pltpu.VMEM((NCHUNK, CHUNK), jnp.int32),
        pltpu.VMEM((CHUNK, EMBED), jnp.float32),
        pltpu.VMEM((CHUNK, EMBED), jnp.float32),
        pltpu.SemaphoreType.DMA,
        pltpu.SemaphoreType.DMA,
        pltpu.SemaphoreType.DMA,
        pltpu.SemaphoreType.DMA,
    ],
)(_sc_body)


def kernel(word, emb):
    idx_rows = word.astype(jnp.int32).reshape(N // CHUNK, CHUNK)
    out = _sc_call(emb, idx_rows)
    return out.reshape(BATCH, HIST, EMBED)


# speedup vs baseline: 3.0430x; 3.0430x over previous
import functools

import jax
import jax.numpy as jnp
from jax import lax
from jax.experimental import pallas as pl
from jax.experimental.pallas import tpu as pltpu
from jax.experimental.pallas import tpu_sc as plsc

VOCAB = 100000
EMBED = 64
BATCH = 4096
HIST = 50
N = BATCH * HIST
NC, NS, L = 2, 16, 16
NW = NC * NS
CHUNK = 128
NPW = N // NW
NCHUNK = NPW // CHUNK
VPR = EMBED // L

C0 = 0.99949463
C1 = -0.491904
C2 = 0.28946795
C3 = -0.13606202
C4 = 0.03216066


def _softplus_chunk(buf):
    @pl.loop(0, CHUNK)
    def _row(i):
        for k in range(VPR):
            sl = (i, pl.ds(k * L, L))
            x = buf[sl]
            r = jnp.maximum(x, 0.0)
            t = jnp.exp(-jnp.abs(x))
            p = C4
            p = p * t + C3
            p = p * t + C2
            p = p * t + C1
            p = p * t + C0
            buf[sl] = r + t * p


def _sc_body(emb_hbm, word_hbm, out_hbm, idx_v, buf0, buf1,
             gsem0, gsem1, ssem0, ssem1):
    c = lax.axis_index("c")
    s = lax.axis_index("s")
    wid = c * NS + s
    row0 = wid * NCHUNK

    pltpu.sync_copy(word_hbm.at[wid], idx_v)

    def start_gather(chunk, buf, sem):
        return pltpu.async_copy(emb_hbm.at[idx_v.at[chunk]], buf, sem)

    def wait_gather(chunk, buf, sem):
        pltpu.make_async_copy(emb_hbm.at[idx_v.at[chunk]], buf, sem).wait()

    def out_slice(chunk):
        return out_hbm.at[pl.ds((row0 + chunk) * CHUNK, CHUNK)]

    def start_store(chunk, buf, sem):
        return pltpu.async_copy(buf, out_slice(chunk), sem)

    def wait_store(chunk, buf, sem):
        pltpu.make_async_copy(buf, out_slice(chunk), sem).wait()

    start_gather(0, buf0, gsem0)

    @pl.loop(0, NCHUNK, step=2)
    def _pair(j):
        c0 = j
        c1 = j + 1

        @pl.when(c0 > 0)
        def _():
            wait_store(c1 - 2, buf1, ssem1)

        start_gather(c1, buf1, gsem1)
        wait_gather(c0, buf0, gsem0)
        _softplus_chunk(buf0)
        start_store(c0, buf0, ssem0)

        @pl.when(c0 + 2 < NCHUNK)
        def _():
            wait_store(c0, buf0, ssem0)
            start_gather(c0 + 2, buf0, gsem0)

        wait_gather(c1, buf1, gsem1)
        _softplus_chunk(buf1)
        start_store(c1, buf1, ssem1)

    wait_store(NCHUNK - 2, buf0, ssem0)
    wait_store(NCHUNK - 1, buf1, ssem1)


_sc_call = functools.partial(
    pl.kernel,
    out_type=jax.ShapeDtypeStruct((N, EMBED), jnp.float32),
    mesh=plsc.VectorSubcoreMesh(
        core_axis_name="c", subcore_axis_name="s",
        num_cores=NC, num_subcores=NS),
    compiler_params=pltpu.CompilerParams(use_tc_tiling_on_sc=False),
    scratch_types=[
        pltpu.VMEM((NCHUNK, CHUNK), jnp.int32),
        pltpu.VMEM((CHUNK, EMBED), jnp.float32),
        pltpu.VMEM((CHUNK, EMBED), jnp.float32),
        pltpu.SemaphoreType.DMA,
        pltpu.SemaphoreType.DMA,
        pltpu.SemaphoreType.DMA,
        pltpu.SemaphoreType.DMA,
    ],
)(_sc_body)


def kernel(word, emb):
    idx_rows = word.astype(jnp.int32).reshape(NW, NCHUNK, CHUNK)
    out = _sc_call(emb, idx_rows)
    return out.reshape(BATCH, HIST, EMBED)


# speedup vs baseline: 3.6347x; 1.1944x over previous
import functools

import jax
import jax.numpy as jnp
from jax import lax
from jax.experimental import pallas as pl
from jax.experimental.pallas import tpu as pltpu
from jax.experimental.pallas import tpu_sc as plsc

VOCAB = 100000
EMBED = 64
BATCH = 4096
HIST = 50
NC, NS, L = 2, 16, 16
NW = NC * NS
ROWS_PW = BATCH // NW
VPR = EMBED // L

C0 = 0.99949463
C1 = -0.491904
C2 = 0.28946795
C3 = -0.13606202
C4 = 0.03216066


def _softplus_buf(buf):
    @pl.loop(0, HIST, unroll=2)
    def _row(i):
        for k in range(VPR):
            sl = (i, pl.ds(k * L, L))
            x = buf[sl]
            r = jnp.maximum(x, 0.0)
            t = jnp.exp(-jnp.abs(x))
            t2 = t * t
            p01 = C1 * t + C0
            p23 = C3 * t + C2
            p = (C4 * t2 + p23) * t2 + p01
            buf[sl] = t * p + r


def _sc_body(emb_hbm, word_hbm, out_hbm, idx_v, buf0, buf1,
             gsem0, gsem1, ssem0, ssem1):
    c = lax.axis_index("c")
    s = lax.axis_index("s")
    wid = c * NS + s
    row0 = wid * ROWS_PW

    pltpu.sync_copy(word_hbm.at[pl.ds(row0, ROWS_PW)], idx_v)

    def start_gather(r, buf, sem):
        return pltpu.async_copy(emb_hbm.at[idx_v.at[r]], buf, sem)

    def wait_gather(r, buf, sem):
        pltpu.make_async_copy(emb_hbm.at[idx_v.at[r]], buf, sem).wait()

    def start_store(r, buf, sem):
        return pltpu.async_copy(buf, out_hbm.at[row0 + r], sem)

    def wait_store(r, buf, sem):
        pltpu.make_async_copy(buf, out_hbm.at[row0 + r], sem).wait()

    start_gather(0, buf0, gsem0)

    @pl.loop(0, ROWS_PW, step=2)
    def _pair(j):
        r0 = j
        r1 = j + 1

        @pl.when(r0 > 0)
        def _():
            wait_store(r1 - 2, buf1, ssem1)

        start_gather(r1, buf1, gsem1)
        wait_gather(r0, buf0, gsem0)
        _softplus_buf(buf0)
        start_store(r0, buf0, ssem0)

        @pl.when(r0 + 2 < ROWS_PW)
        def _():
            wait_store(r0, buf0, ssem0)
            start_gather(r0 + 2, buf0, gsem0)

        wait_gather(r1, buf1, gsem1)
        _softplus_buf(buf1)
        start_store(r1, buf1, ssem1)

    wait_store(ROWS_PW - 2, buf0, ssem0)
    wait_store(ROWS_PW - 1, buf1, ssem1)


_sc_call = functools.partial(
    pl.kernel,
    out_type=jax.ShapeDtypeStruct((BATCH, HIST, EMBED), jnp.float32),
    mesh=plsc.VectorSubcoreMesh(
        core_axis_name="c", subcore_axis_name="s",
        num_cores=NC, num_subcores=NS),
    compiler_params=pltpu.CompilerParams(use_tc_tiling_on_sc=False),
    scratch_types=[
        pltpu.VMEM((ROWS_PW, HIST), jnp.int32),
        pltpu.VMEM((HIST, EMBED), jnp.float32),
        pltpu.VMEM((HIST, EMBED), jnp.float32),
        pltpu.SemaphoreType.DMA,
        pltpu.SemaphoreType.DMA,
        pltpu.SemaphoreType.DMA,
        pltpu.SemaphoreType.DMA,
    ],
)(_sc_body)


def kernel(word, emb):
    return _sc_call(emb, word.astype(jnp.int32))


# speedup vs baseline: 3.9404x; 1.0841x over previous
import functools

import jax
import jax.numpy as jnp
from jax import lax
from jax.experimental import pallas as pl
from jax.experimental.pallas import tpu as pltpu
from jax.experimental.pallas import tpu_sc as plsc

VOCAB = 100000
EMBED = 64
BATCH = 4096
HIST = 50
NC, NS, L = 2, 16, 16
NW = NC * NS
ROWS_PW = BATCH // NW
VPR = EMBED // L

C0 = 0.99949463
C1 = -0.491904
C2 = 0.28946795
C3 = -0.13606202
C4 = 0.03216066


def _softplus_buf(buf):
    @pl.loop(0, HIST, unroll=2)
    def _row(i):
        for k in range(VPR):
            sl = (i, pl.ds(k * L, L))
            x = buf[sl]
            r = jnp.maximum(x, 0.0)
            t = jnp.exp(-jnp.abs(x))
            t2 = t * t
            p01 = C1 * t + C0
            p23 = C3 * t + C2
            p = (C4 * t2 + p23) * t2 + p01
            buf[sl] = t * p + r


NB = 4


def _sc_body(emb_hbm, word_hbm, out_hbm, idx_v, buf0, buf1, buf2, buf3,
             gsem0, gsem1, gsem2, gsem3, ssem0, ssem1, ssem2, ssem3):
    bufs = (buf0, buf1, buf2, buf3)
    gsems = (gsem0, gsem1, gsem2, gsem3)
    ssems = (ssem0, ssem1, ssem2, ssem3)
    c = lax.axis_index("c")
    s = lax.axis_index("s")
    wid = c * NS + s
    row0 = wid * ROWS_PW

    pltpu.sync_copy(word_hbm.at[pl.ds(row0, ROWS_PW)], idx_v)

    def start_gather(r, b):
        return pltpu.async_copy(emb_hbm.at[idx_v.at[r]], bufs[b], gsems[b])

    def wait_gather(r, b):
        pltpu.make_async_copy(emb_hbm.at[idx_v.at[r]], bufs[b], gsems[b]).wait()

    def start_store(r, b):
        return pltpu.async_copy(bufs[b], out_hbm.at[row0 + r], ssems[b])

    def wait_store(r, b):
        pltpu.make_async_copy(bufs[b], out_hbm.at[row0 + r], ssems[b]).wait()

    start_gather(0, 0)
    start_gather(1, 1)

    @pl.loop(0, ROWS_PW, step=NB)
    def _ring(j):
        for b in range(NB):
            r = j + b

            bn = (b + 2) % NB

            @pl.when(r + 2 < ROWS_PW)
            def _():
                @pl.when(r >= 2)
                def _():
                    wait_store(r - 2, bn)

                start_gather(r + 2, bn)

            wait_gather(r, b)
            _softplus_buf(bufs[b])
            start_store(r, b)

    for r in range(ROWS_PW - NB, ROWS_PW):
        wait_store(r, r % NB)


_sc_call = functools.partial(
    pl.kernel,
    out_type=jax.ShapeDtypeStruct((BATCH, HIST, EMBED), jnp.float32),
    mesh=plsc.VectorSubcoreMesh(
        core_axis_name="c", subcore_axis_name="s",
        num_cores=NC, num_subcores=NS),
    compiler_params=pltpu.CompilerParams(use_tc_tiling_on_sc=False),
    scratch_types=[
        pltpu.VMEM((ROWS_PW, HIST), jnp.int32),
    ] + [pltpu.VMEM((HIST, EMBED), jnp.float32)] * NB
      + [pltpu.SemaphoreType.DMA] * (2 * NB),
)(_sc_body)


def kernel(word, emb):
    return _sc_call(emb, word.astype(jnp.int32))
